# Initial kernel scaffold; baseline (speedup 1.0000x reference)
import functools

import jax
import jax.numpy as jnp
from jax import lax
from jax.experimental import pallas as pl
from jax.experimental.pallas import tpu as pltpu
from jax.experimental.pallas import tpu_sc as plsc

N = 10000
E = 320000
D = 128
G = 64

NC = 2
NS = 16
NW = NC * NS
EW = E // NW
K = 80
NI = EW // K
RPT = N // NS
ZR = 125
ZCH = RPT // ZR


def _sc_body(er_hbm, x_hbm, out_hbm, sidx, didx, rows0, rows1, zbuf,
             sem0, sem1, shared):
    c = lax.axis_index("c")
    s = lax.axis_index("s")
    w = c * NS + s

    zero = jnp.zeros((16,), jnp.float32)

    def _zfill(i, carry):
        zbuf[i // 8, pl.ds((i % 8) * 16, 16)] = zero
        return carry

    lax.fori_loop(0, ZR * 8, _zfill, 0)
    for z in range(ZCH):
        pltpu.sync_copy(zbuf, shared.at[pl.ds(s * RPT + z * ZR, ZR)])
    plsc.subcore_barrier()

    pltpu.sync_copy(er_hbm.at[0, w], sidx)
    pltpu.sync_copy(er_hbm.at[1, w], didx)

    def _gather(i, buf, sem):
        pltpu.async_copy(x_hbm.at[sidx.at[i]], buf, sem)

    def _wait(i, buf, sem):
        pltpu.make_async_copy(x_hbm.at[sidx.at[i]], buf, sem).wait()

    def _scatter(i, buf):
        pltpu.sync_copy(buf, shared.at[didx.at[i]], add=True)

    _gather(0, rows0, sem0)

    def _eloop(j, carry):
        i0 = 2 * j
        _gather(i0 + 1, rows1, sem1)
        _wait(i0, rows0, sem0)
        _scatter(i0, rows0)

        @pl.when(i0 + 2 < NI)
        def _():
            _gather(i0 + 2, rows0, sem0)

        _wait(i0 + 1, rows1, sem1)
        _scatter(i0 + 1, rows1)
        return carry

    lax.fori_loop(0, NI // 2, _eloop, 0)
    _wait(NI - 1, rows0, sem0)
    _scatter(NI - 1, rows0)

    plsc.subcore_barrier()
    pltpu.sync_copy(shared.at[pl.ds(s * RPT, RPT)],
                    out_hbm.at[c, pl.ds(s * RPT, RPT)])


@jax.jit
def _sc_aggregate(edge_index, x):
    er = edge_index.reshape(2, NW, NI, K)
    mesh = plsc.VectorSubcoreMesh(core_axis_name="c", subcore_axis_name="s")
    fn = pl.kernel(
        _sc_body,
        out_type=jax.ShapeDtypeStruct((NC, N, D), jnp.float32),
        mesh=mesh,
        scratch_types=[
            pltpu.VMEM((NI, K), jnp.int32),
            pltpu.VMEM((NI, K), jnp.int32),
            pltpu.VMEM((K, D), jnp.float32),
            pltpu.VMEM((K, D), jnp.float32),
            pltpu.VMEM((ZR, D), jnp.float32),
            pltpu.SemaphoreType.DMA,
            pltpu.SemaphoreType.DMA,
            pltpu.VMEM_SHARED((N, D), jnp.float32),
        ],
    )
    return fn(er, x)


R = 2000
NB = N // R


def _tc_body(xb, ab, bb, W1b, b1b, W2b, b2b, W3b, b3b, outb):
    i = pl.program_id(0)
    h = xb[...] + ab[0] + ab[1]
    h = jnp.maximum(jnp.dot(h, W1b[...], preferred_element_type=jnp.float32)
                    + b1b[...], 0.0)
    h = jnp.maximum(jnp.dot(h, W2b[...], preferred_element_type=jnp.float32)
                    + b2b[...], 0.0)
    o = jnp.dot(h, W3b[...], preferred_element_type=jnp.float32) + b3b[...]
    gids = lax.broadcasted_iota(jnp.int32, (G, R), 0)
    onehot = (bb[0] == gids).astype(jnp.float32)
    seg = jnp.dot(onehot, o, preferred_element_type=jnp.float32)

    @pl.when(i == 0)
    def _():
        outb[...] = seg

    @pl.when(i > 0)
    def _():
        outb[...] += seg


@jax.jit
def _tc_mlp_pool(x, agg, batch, W1, b1, W2, b2, W3, b3):
    O = W3.shape[1]
    b3d = batch.reshape(NB, 1, R)
    full = lambda *_: (0, 0)
    out = pl.pallas_call(
        _tc_body,
        grid=(NB,),
        in_specs=[
            pl.BlockSpec((R, D), lambda i: (i, 0)),
            pl.BlockSpec((NC, R, D), lambda i: (0, i, 0)),
            pl.BlockSpec((1, 1, R), lambda i: (i, 0, 0)),
            pl.BlockSpec((D, D), full),
            pl.BlockSpec((1, D), full),
            pl.BlockSpec((D, D), full),
            pl.BlockSpec((1, D), full),
            pl.BlockSpec((D, O), full),
            pl.BlockSpec((1, O), full),
        ],
        out_specs=pl.BlockSpec((G, O), full),
        out_shape=jax.ShapeDtypeStruct((G, O), jnp.float32),
    )(x, agg, b3d, W1, b1.reshape(1, D), W2, b2.reshape(1, D),
      W3, b3.reshape(1, O))
    return out


def kernel(x, edge_index, batch, W1, b1, W2, b2, W3, b3):
    agg = _sc_aggregate(edge_index, x)
    return _tc_mlp_pool(x, agg, batch, W1, b1, W2, b2, W3, b3)


# speedup vs baseline: 12.2419x; 12.2419x over previous
import functools

import jax
import jax.numpy as jnp
from jax import lax
from jax.experimental import pallas as pl
from jax.experimental.pallas import tpu as pltpu
from jax.experimental.pallas import tpu_sc as plsc

N = 10000
E = 320000
D = 128
G = 64

NC = 2
NS = 16
NW = NC * NS
EW = E // NW
K = 128
CPB = 8
NBLK = 10
EWP = NBLK * CPB * K
PADW = EWP - EW
NP = 10240
RPT = NP // NS
ZCH = RPT // K


def _sc_body(er_hbm, x_hbm, out_hbm, sidx, didx, rows0, rows1,
             sem0, sem1, shared):
    c = lax.axis_index("c")
    s = lax.axis_index("s")
    w = c * NS + s

    zero = jnp.zeros((16,), jnp.float32)

    def _zfill(i, carry):
        rows0[i // 8, pl.ds((i % 8) * 16, 16)] = zero
        return carry

    lax.fori_loop(0, K * 8, _zfill, 0)
    for z in range(ZCH):
        pltpu.sync_copy(rows0, shared.at[pl.ds(s * RPT + z * K, K)])
    plsc.subcore_barrier()

    bufs = (rows0, rows1)
    sems = (sem0, sem1)

    def _gather(j, buf, sem):
        pltpu.async_copy(x_hbm.at[sidx.at[j]], buf, sem)

    def _wait(j, buf, sem):
        pltpu.make_async_copy(x_hbm.at[sidx.at[j]], buf, sem).wait()

    def _scatter(j, buf):
        pltpu.sync_copy(buf, shared.at[didx.at[j]], add=True)

    def _block(blk, carry):
        pltpu.sync_copy(er_hbm.at[0, w, blk], sidx)
        pltpu.sync_copy(er_hbm.at[1, w, blk], didx)
        _gather(0, rows0, sem0)
        for j in range(CPB - 1):
            _gather(j + 1, bufs[(j + 1) % 2], sems[(j + 1) % 2])
            _wait(j, bufs[j % 2], sems[j % 2])
            _scatter(j, bufs[j % 2])
        _wait(CPB - 1, bufs[(CPB - 1) % 2], sems[(CPB - 1) % 2])
        _scatter(CPB - 1, bufs[(CPB - 1) % 2])
        return carry

    lax.fori_loop(0, NBLK, _block, 0)

    plsc.subcore_barrier()
    pltpu.sync_copy(shared.at[pl.ds(s * RPT, RPT)],
                    out_hbm.at[c, pl.ds(s * RPT, RPT)])


@jax.jit
def _sc_aggregate(edge_index, x):
    ei = edge_index.reshape(2, NW, EW)
    j = jnp.arange(PADW, dtype=jnp.int32)
    wv = jnp.arange(NW, dtype=jnp.int32)[:, None]
    pad_src = (wv * 317 + j * 13) % N
    pad_dst = jnp.broadcast_to(N + j, (NW, PADW)).astype(jnp.int32)
    er = jnp.concatenate(
        [ei, jnp.stack([pad_src, pad_dst])], axis=2
    ).reshape(2, NW, NBLK, CPB, K)
    mesh = plsc.VectorSubcoreMesh(core_axis_name="c", subcore_axis_name="s")
    fn = pl.kernel(
        _sc_body,
        out_type=jax.ShapeDtypeStruct((NC, NP, D), jnp.float32),
        mesh=mesh,
        scratch_types=[
            pltpu.VMEM((CPB, K), jnp.int32),
            pltpu.VMEM((CPB, K), jnp.int32),
            pltpu.VMEM((K, D), jnp.float32),
            pltpu.VMEM((K, D), jnp.float32),
            pltpu.SemaphoreType.DMA,
            pltpu.SemaphoreType.DMA,
            pltpu.VMEM_SHARED((NP, D), jnp.float32),
        ],
    )
    return fn(er, x)


R = 2000
NB = N // R


def _tc_body(xb, ab, bb, W1b, b1b, W2b, b2b, W3b, b3b, outb):
    i = pl.program_id(0)
    h = xb[...] + ab[0] + ab[1]
    h = jnp.maximum(jnp.dot(h, W1b[...], preferred_element_type=jnp.float32)
                    + b1b[...], 0.0)
    h = jnp.maximum(jnp.dot(h, W2b[...], preferred_element_type=jnp.float32)
                    + b2b[...], 0.0)
    o = jnp.dot(h, W3b[...], preferred_element_type=jnp.float32) + b3b[...]
    gids = lax.broadcasted_iota(jnp.int32, (G, R), 0)
    onehot = (bb[0] == gids).astype(jnp.float32)
    seg = jnp.dot(onehot, o, preferred_element_type=jnp.float32)

    @pl.when(i == 0)
    def _():
        outb[...] = seg

    @pl.when(i > 0)
    def _():
        outb[...] += seg


@jax.jit
def _tc_mlp_pool(x, agg, batch, W1, b1, W2, b2, W3, b3):
    O = W3.shape[1]
    b3d = batch.reshape(NB, 1, R)
    full = lambda *_: (0, 0)
    out = pl.pallas_call(
        _tc_body,
        grid=(NB,),
        in_specs=[
            pl.BlockSpec((R, D), lambda i: (i, 0)),
            pl.BlockSpec((NC, R, D), lambda i: (0, i, 0)),
            pl.BlockSpec((1, 1, R), lambda i: (i, 0, 0)),
            pl.BlockSpec((D, D), full),
            pl.BlockSpec((1, D), full),
            pl.BlockSpec((D, D), full),
            pl.BlockSpec((1, D), full),
            pl.BlockSpec((D, O), full),
            pl.BlockSpec((1, O), full),
        ],
        out_specs=pl.BlockSpec((G, O), full),
        out_shape=jax.ShapeDtypeStruct((G, O), jnp.float32),
    )(x, agg, b3d, W1, b1.reshape(1, D), W2, b2.reshape(1, D),
      W3, b3.reshape(1, O))
    return out


def kernel(x, edge_index, batch, W1, b1, W2, b2, W3, b3):
    agg = _sc_aggregate(edge_index, x)
    return _tc_mlp_pool(x, agg, batch, W1, b1, W2, b2, W3, b3)


# speedup vs baseline: 13.1523x; 1.0744x over previous
import functools

import jax
import jax.numpy as jnp
from jax import lax
from jax.experimental import pallas as pl
from jax.experimental.pallas import tpu as pltpu
from jax.experimental.pallas import tpu_sc as plsc

N = 10000
E = 320000
D = 128
G = 64

NC = 2
NS = 16
NW = NC * NS
EW = E // NW
K = 128
CPB = 8
NBLK = 10
EWP = NBLK * CPB * K
PADW = EWP - EW
NP = 10240
RPT = NP // NS
ZCH = RPT // K


def _sc_body(er_hbm, x_hbm, out_hbm, sidx, didx, rows0, rows1,
             sem0, sem1, isem, shared):
    c = lax.axis_index("c")
    s = lax.axis_index("s")
    w = c * NS + s

    zero = jnp.zeros((16,), jnp.float32)

    def _zfill(i, carry):
        rows0[i // 8, pl.ds((i % 8) * 16, 16)] = zero
        return carry

    lax.fori_loop(0, K * 8, _zfill, 0)
    for z in range(ZCH):
        pltpu.sync_copy(rows0, shared.at[pl.ds(s * RPT + z * K, K)])
    plsc.subcore_barrier()

    bufs = (rows0, rows1)
    sems = (sem0, sem1)

    def _gather(ph, j, buf, sem):
        pltpu.async_copy(x_hbm.at[sidx.at[ph, j]], buf, sem)

    def _wait(ph, j, buf, sem):
        pltpu.make_async_copy(x_hbm.at[sidx.at[ph, j]], buf, sem).wait()

    def _scatter(ph, j, buf):
        pltpu.sync_copy(buf, shared.at[didx.at[ph, j]], add=True)

    pltpu.sync_copy(er_hbm.at[0, w, 0], sidx.at[0])
    pltpu.sync_copy(er_hbm.at[1, w, 0], didx.at[0])

    def _block(blk, carry):
        ph = lax.rem(blk, 2)

        @pl.when(blk > 0)
        def _():
            pltpu.make_async_copy(er_hbm.at[0, w, blk], sidx.at[ph],
                                  isem).wait()
            pltpu.make_async_copy(er_hbm.at[1, w, blk], didx.at[ph],
                                  isem).wait()

        @pl.when(blk < NBLK - 1)
        def _():
            pltpu.async_copy(er_hbm.at[0, w, blk + 1], sidx.at[1 - ph], isem)
            pltpu.async_copy(er_hbm.at[1, w, blk + 1], didx.at[1 - ph], isem)

        _gather(ph, 0, rows0, sem0)
        for j in range(CPB - 1):
            _gather(ph, j + 1, bufs[(j + 1) % 2], sems[(j + 1) % 2])
            _wait(ph, j, bufs[j % 2], sems[j % 2])
            _scatter(ph, j, bufs[j % 2])
        _wait(ph, CPB - 1, bufs[(CPB - 1) % 2], sems[(CPB - 1) % 2])
        _scatter(ph, CPB - 1, bufs[(CPB - 1) % 2])
        return carry

    lax.fori_loop(0, NBLK, _block, 0)

    plsc.subcore_barrier()
    pltpu.sync_copy(shared.at[pl.ds(s * RPT, RPT)],
                    out_hbm.at[c, pl.ds(s * RPT, RPT)])


@jax.jit
def _sc_aggregate(edge_index, x):
    ei = edge_index.reshape(2, NW, EW)
    j = jnp.arange(PADW, dtype=jnp.int32)
    wv = jnp.arange(NW, dtype=jnp.int32)[:, None]
    pad_src = (wv * 317 + j * 13) % N
    pad_dst = jnp.broadcast_to(N + j, (NW, PADW)).astype(jnp.int32)
    er = jnp.concatenate(
        [ei, jnp.stack([pad_src, pad_dst])], axis=2
    ).reshape(2, NW, NBLK, CPB, K)
    mesh = plsc.VectorSubcoreMesh(core_axis_name="c", subcore_axis_name="s")
    fn = pl.kernel(
        _sc_body,
        out_type=jax.ShapeDtypeStruct((NC, NP, D), jnp.float32),
        mesh=mesh,
        scratch_types=[
            pltpu.VMEM((2, CPB, K), jnp.int32),
            pltpu.VMEM((2, CPB, K), jnp.int32),
            pltpu.VMEM((K, D), jnp.float32),
            pltpu.VMEM((K, D), jnp.float32),
            pltpu.SemaphoreType.DMA,
            pltpu.SemaphoreType.DMA,
            pltpu.SemaphoreType.DMA,
            pltpu.VMEM_SHARED((NP, D), jnp.float32),
        ],
    )
    return fn(er, x)


R = 2000
NB = N // R


def _tc_body(xb, ab, bb, W1b, b1b, W2b, b2b, W3b, b3b, outb):
    i = pl.program_id(0)
    h = xb[...] + ab[0] + ab[1]
    h = jnp.maximum(jnp.dot(h, W1b[...], preferred_element_type=jnp.float32)
                    + b1b[...], 0.0)
    h = jnp.maximum(jnp.dot(h, W2b[...], preferred_element_type=jnp.float32)
                    + b2b[...], 0.0)
    o = jnp.dot(h, W3b[...], preferred_element_type=jnp.float32) + b3b[...]
    gids = lax.broadcasted_iota(jnp.int32, (G, R), 0)
    onehot = (bb[0] == gids).astype(jnp.float32)
    seg = jnp.dot(onehot, o, preferred_element_type=jnp.float32)

    @pl.when(i == 0)
    def _():
        outb[...] = seg

    @pl.when(i > 0)
    def _():
        outb[...] += seg


@jax.jit
def _tc_mlp_pool(x, agg, batch, W1, b1, W2, b2, W3, b3):
    O = W3.shape[1]
    b3d = batch.reshape(NB, 1, R)
    full = lambda *_: (0, 0)
    out = pl.pallas_call(
        _tc_body,
        grid=(NB,),
        in_specs=[
            pl.BlockSpec((R, D), lambda i: (i, 0)),
            pl.BlockSpec((NC, R, D), lambda i: (0, i, 0)),
            pl.BlockSpec((1, 1, R), lambda i: (i, 0, 0)),
            pl.BlockSpec((D, D), full),
            pl.BlockSpec((1, D), full),
            pl.BlockSpec((D, D), full),
            pl.BlockSpec((1, D), full),
            pl.BlockSpec((D, O), full),
            pl.BlockSpec((1, O), full),
        ],
        out_specs=pl.BlockSpec((G, O), full),
        out_shape=jax.ShapeDtypeStruct((G, O), jnp.float32),
    )(x, agg, b3d, W1, b1.reshape(1, D), W2, b2.reshape(1, D),
      W3, b3.reshape(1, O))
    return out


def kernel(x, edge_index, batch, W1, b1, W2, b2, W3, b3):
    agg = _sc_aggregate(edge_index, x)
    return _tc_mlp_pool(x, agg, batch, W1, b1, W2, b2, W3, b3)
